# baseline (speedup 1.0000x reference)
import functools

import jax
import jax.numpy as jnp
from jax import lax
from jax.experimental import pallas as pl
from jax.experimental.pallas import tpu as pltpu
from jax.experimental.pallas import tpu_sc as plsc

S = 19
N = 2097152
NC = 2
NS = 16
NW = NC * NS
R = N // NW
CHC = 1024
NB = 4
NCH = R // CHC
G = CHC // 16

_mesh = plsc.VectorSubcoreMesh(core_axis_name="c", subcore_axis_name="s")

_scratch = (
    [pltpu.VMEM((CHC,), jnp.float32) for _ in range(NB)]
    + [pltpu.VMEM((S, CHC), jnp.float32) for _ in range(NB)]
    + [pltpu.VMEM((CHC,), jnp.int32) for _ in range(NB)]
    + [pltpu.SemaphoreType.DMA for _ in range(2 * NB)]
)


@functools.partial(
    pl.kernel,
    mesh=_mesh,
    out_type=jax.ShapeDtypeStruct((S, N), jnp.float32),
    scratch_types=_scratch,
    compiler_params=pltpu.CompilerParams(needs_layout_passes=False),
)
def _sc_twohot(x_hbm, out_hbm, *refs):
    xvs = refs[0:NB]
    ovs = refs[NB:2 * NB]
    fvs = refs[2 * NB:3 * NB]
    sems = refs[3 * NB:4 * NB]
    semxs = refs[4 * NB:5 * NB]

    wid = lax.axis_index("s") * NC + lax.axis_index("c")
    base_col = wid * R

    lanes = lax.iota(jnp.int32, 16)
    zf = jnp.zeros((16,), jnp.float32)
    f_init = jnp.full((16,), S, jnp.int32)

    def init_body(i, _):
        c16 = i * 16 + lanes
        for j in range(S):
            cj = jnp.full((16,), j, jnp.int32)
            for b in range(NB):
                plsc.store_scatter(ovs[b], [cj, c16], zf)
        for b in range(NB):
            fvs[b][pl.ds(i * 16, 16)] = f_init
        return 0
    lax.fori_loop(0, CHC // 16, init_body, 0)

    pltpu.async_copy(x_hbm.at[pl.ds(base_col, CHC)], xvs[0], semxs[0])

    def outer(o, _):
        for b in range(NB):
            xv, ov, fv, sem, semx = xvs[b], ovs[b], fvs[b], sems[b], semxs[b]
            bn = (b + 1) % NB
            c = o * NB + b
            col0 = base_col + c * CHC
            @pl.when(o > 0)
            def _wait():
                pltpu.make_async_copy(
                    ov, out_hbm.at[:, pl.ds(0, CHC)], sem
                ).wait()

            pltpu.make_async_copy(x_hbm.at[pl.ds(0, CHC)], xv, semx).wait()

            @pl.when(c + 1 < NCH)
            def _prefetch():
                pltpu.async_copy(
                    x_hbm.at[pl.ds(col0 + CHC, CHC)], xvs[bn], semxs[bn]
                )

            @plsc.parallel_loop(0, G, step=1, unroll=4)
            def group(g):
                cols = g * 16 + lanes
                fold = fv[pl.ds(g * 16, 16)]
                plsc.store_scatter(ov, [fold, cols], zf, mask=fold < S)
                plsc.store_scatter(ov, [fold + 1, cols], zf, mask=fold + 1 < S)

                xx = xv[pl.ds(g * 16, 16)]
                ax = jnp.abs(xx) + 1.0
                ib = plsc.bitcast(ax, jnp.int32)
                z = plsc.bitcast(
                    jnp.int32(0x5F3759DF) - jnp.right_shift(ib, 1), jnp.float32
                )
                h = 0.5 * ax
                z = z * (1.5 - h * z * z)
                z = z * (1.5 - h * z * z)
                z = z * (1.5 - h * z * z)
                y = ax * z
                tv = jnp.sign(xx) * (y - 1.0 + 0.001 * xx)
                tv = jnp.clip(tv, 0.0, float(S))
                fi = tv.astype(jnp.int32)
                r = tv - fi.astype(jnp.float32)

                plsc.store_scatter(ov, [fi, cols], 1.0 - r, mask=fi < S)
                plsc.store_scatter(ov, [fi + 1, cols], r, mask=fi + 1 < S)
                fv[pl.ds(g * 16, 16)] = fi

            pltpu.async_copy(ov, out_hbm.at[:, pl.ds(col0, CHC)], sem)
        return 0

    lax.fori_loop(0, NCH // NB, outer, 0)

    for b in range(NB):
        pltpu.make_async_copy(ovs[b], out_hbm.at[:, pl.ds(0, CHC)], sems[b]).wait()


def kernel(target_value):
    return _sc_twohot(target_value).T


# speedup vs baseline: 1.0471x; 1.0471x over previous
import functools

import jax
import jax.numpy as jnp
from jax import lax
from jax.experimental import pallas as pl
from jax.experimental.pallas import tpu as pltpu
from jax.experimental.pallas import tpu_sc as plsc

S = 19
N = 2097152
NC = 2
NS = 16
NW = NC * NS
R = N // NW
CHC = 2048
NCH = R // CHC
G = CHC // 16

_mesh = plsc.VectorSubcoreMesh(core_axis_name="c", subcore_axis_name="s")


@functools.partial(
    pl.kernel,
    mesh=_mesh,
    out_type=jax.ShapeDtypeStruct((S, N), jnp.float32),
    scratch_types=[
        pltpu.VMEM((CHC,), jnp.float32),
        pltpu.VMEM((CHC,), jnp.float32),
        pltpu.VMEM((S, CHC), jnp.float32),
        pltpu.VMEM((S, CHC), jnp.float32),
        pltpu.VMEM((CHC,), jnp.int32),
        pltpu.VMEM((CHC,), jnp.int32),
        pltpu.SemaphoreType.DMA,
        pltpu.SemaphoreType.DMA,
        pltpu.SemaphoreType.DMA,
        pltpu.SemaphoreType.DMA,
    ],
    compiler_params=pltpu.CompilerParams(needs_layout_passes=False),
)
def _sc_twohot(
    x_hbm, out_hbm, xv0, xv1, ov0, ov1, fv0, fv1, sem0, sem1, semx0, semx1
):
    wid = lax.axis_index("s") * NC + lax.axis_index("c")
    base_col = wid * R

    lanes = lax.iota(jnp.int32, 16)
    zf = jnp.zeros((16,), jnp.float32)
    f_init = jnp.full((16,), S, jnp.int32)

    bufs = ((xv0, ov0, fv0, sem0, semx0), (xv1, ov1, fv1, sem1, semx1))

    def init_body(i, _):
        c16 = i * 16 + lanes
        for j in range(S):
            cj = jnp.full((16,), j, jnp.int32)
            plsc.store_scatter(ov0, [cj, c16], zf)
            plsc.store_scatter(ov1, [cj, c16], zf)
        fv0[pl.ds(i * 16, 16)] = f_init
        fv1[pl.ds(i * 16, 16)] = f_init
        return 0
    lax.fori_loop(0, CHC // 16, init_body, 0)

    pltpu.async_copy(x_hbm.at[pl.ds(base_col, CHC)], xv0, semx0)

    def outer(o, _):
        for b in range(2):
            xv, ov, fv, sem, semx = bufs[b]
            xvn, _, _, _, semxn = bufs[1 - b]
            c = o * 2 + b
            col0 = base_col + c * CHC
            @pl.when(o > 0)
            def _wait():
                pltpu.make_async_copy(
                    ov, out_hbm.at[:, pl.ds(0, CHC)], sem
                ).wait()

            pltpu.make_async_copy(x_hbm.at[pl.ds(0, CHC)], xv, semx).wait()

            @pl.when(c + 1 < NCH)
            def _prefetch():
                pltpu.async_copy(
                    x_hbm.at[pl.ds(col0 + CHC, CHC)], xvn, semxn
                )

            @plsc.parallel_loop(0, G, step=1, unroll=4)
            def group(g):
                cols = g * 16 + lanes
                fold = fv[pl.ds(g * 16, 16)]
                plsc.store_scatter(ov, [fold, cols], zf, mask=fold < S)
                plsc.store_scatter(ov, [fold + 1, cols], zf, mask=fold + 1 < S)

                xx = xv[pl.ds(g * 16, 16)]
                ax = jnp.abs(xx) + 1.0
                ib = plsc.bitcast(ax, jnp.int32)
                z = plsc.bitcast(
                    jnp.int32(0x5F3759DF) - jnp.right_shift(ib, 1), jnp.float32
                )
                h = 0.5 * ax
                z = z * (1.5 - h * z * z)
                z = z * (1.5 - h * z * z)
                z = z * (1.5 - h * z * z)
                y = ax * z
                tv = jnp.sign(xx) * (y - 1.0 + 0.001 * xx)
                tv = jnp.clip(tv, 0.0, float(S))
                fi = tv.astype(jnp.int32)
                r = tv - fi.astype(jnp.float32)

                plsc.store_scatter(ov, [fi, cols], 1.0 - r, mask=fi < S)
                plsc.store_scatter(ov, [fi + 1, cols], r, mask=fi + 1 < S)
                fv[pl.ds(g * 16, 16)] = fi

            pltpu.async_copy(ov, out_hbm.at[:, pl.ds(col0, CHC)], sem)
        return 0

    lax.fori_loop(0, NCH // 2, outer, 0)

    for b in range(2):
        _, ov, _, sem, _ = bufs[b]
        pltpu.make_async_copy(ov, out_hbm.at[:, pl.ds(0, CHC)], sem).wait()


def kernel(target_value):
    return _sc_twohot(target_value).T
